# Initial kernel scaffold; baseline (speedup 1.0000x reference)
import jax
import jax.numpy as jnp
from jax.experimental import pallas as pl


def kernel(x, pointer_addresses, W, b):
    raise NotImplementedError("write your pallas kernel here")


# speedup vs baseline: 1.0430x; 1.0430x over previous
import jax
import jax.numpy as jnp
from jax.experimental import pallas as pl
from jax.experimental.pallas import tpu as pltpu

B = 128
D_MODEL = 768
VOCAB = 32000
E = 8
VBLK = 1280
V_TILES = VOCAB // VBLK


def _moe_body(p_ref, x_ref, w_ref, b_ref, o_ref):
    e = pl.program_id(1)
    mask = (p_ref[:] % E) == e
    acc = jax.lax.dot_general(
        x_ref[:], w_ref[0],
        dimension_numbers=(((1,), (1,)), ((), ())),
        preferred_element_type=jnp.float32,
    )
    acc = acc + b_ref[0]

    @pl.when(e == 0)
    def _():
        o_ref[:] = jnp.where(mask, acc, jnp.zeros_like(acc))

    @pl.when(e != 0)
    def _():
        o_ref[:] = jnp.where(mask, acc, o_ref[:])


def kernel(x, pointer_addresses, W, b):
    p2d = pointer_addresses.reshape(B, 1).astype(jnp.int32)
    out = pl.pallas_call(
        _moe_body,
        grid=(V_TILES, E),
        in_specs=[
            pl.BlockSpec((B, 1), lambda v, e: (0, 0)),
            pl.BlockSpec((B, D_MODEL), lambda v, e: (0, 0)),
            pl.BlockSpec((1, VBLK, D_MODEL), lambda v, e: (e, v, 0)),
            pl.BlockSpec((1, 1, VBLK), lambda v, e: (e, 0, v)),
        ],
        out_specs=pl.BlockSpec((B, VBLK), lambda v, e: (0, v)),
        out_shape=jax.ShapeDtypeStruct((B, VOCAB), jnp.float32),
        compiler_params=pltpu.CompilerParams(
            dimension_semantics=("arbitrary", "arbitrary"),
        ),
    )(p2d, x, W, b.reshape(E, 1, VOCAB))
    return out
